# baseline (speedup 1.0000x reference)
import functools

import jax
import jax.numpy as jnp
from jax import lax
from jax.experimental import pallas as pl
from jax.experimental.pallas import tpu as pltpu
from jax.experimental.pallas import tpu_sc as plsc

B = 1024
C = 256
K = 256
CT = 160
CS = C - CT
NW = 32
NG = 8
BPW = B // NG
WPW = BPW // 2
NQ = NW // NG
CPW = CS // NQ
NV = WPW // 16
KU = 16
TCU = 8

_HIMASK = -65536


def _widen(word):
    lo = lax.bitcast_convert_type(lax.shift_left(word, 16), jnp.float32)
    hi = lax.bitcast_convert_type(
        lax.bitwise_and(word, jnp.int32(_HIMASK)), jnp.float32)
    return lo, hi


def _sc_body(p1t_hbm, p2t_hbm, m1_hbm, m2_hbm, out_hbm,
             p1_v, p2_v, m1_v, m2_v, acc_v):
    cid = lax.axis_index("c")
    sid = lax.axis_index("s")
    wid = sid * 2 + cid
    g = lax.rem(wid, NG)
    q = lax.div(wid, NG)
    b0 = g * BPW
    c0 = q * CPW

    pltpu.sync_copy(p1t_hbm.at[g], p1_v)
    pltpu.sync_copy(p2t_hbm.at[g], p2_v)
    pltpu.sync_copy(m1_hbm.at[pl.ds(c0, CPW), :], m1_v)
    pltpu.sync_copy(m2_hbm.at[pl.ds(c0, CPW), :], m2_v)

    def c_body(ci, carry):
        def k_body(kk, accs):
            accs = list(accs)
            kbase = kk * KU
            iv1 = m1_v[ci, pl.ds(kbase, KU)]
            iv2 = m2_v[ci, pl.ds(kbase, KU)]
            for u in range(KU):
                i1 = iv1[u]
                i2 = iv2[u]
                for h in range(NV):
                    a_lo, a_hi = _widen(p1_v[i1, pl.ds(h * 16, 16)])
                    b_lo, b_hi = _widen(p2_v[i2, pl.ds(h * 16, 16)])
                    accs[2 * h] = accs[2 * h] + a_lo * b_lo
                    accs[2 * h + 1] = accs[2 * h + 1] + a_hi * b_hi
            return tuple(accs)

        z = jnp.zeros((16,), jnp.float32)
        accs = lax.fori_loop(0, K // KU, k_body, (z,) * (2 * NV))
        for h in range(NV):
            acc_v[ci, pl.ds(h * 32, 16)] = accs[2 * h]
            acc_v[ci, pl.ds(h * 32 + 16, 16)] = accs[2 * h + 1]
        return carry

    lax.fori_loop(0, CPW, c_body, 0)
    pltpu.sync_copy(acc_v, out_hbm.at[pl.ds(c0, CPW), pl.ds(b0, BPW)])


_sc_call = functools.partial(
    pl.kernel,
    mesh=plsc.VectorSubcoreMesh(core_axis_name="c", subcore_axis_name="s"),
    out_type=jax.ShapeDtypeStruct((CS, B), jnp.float32),
    scratch_types=[
        pltpu.VMEM((C, WPW), jnp.int32),
        pltpu.VMEM((C, WPW), jnp.int32),
        pltpu.VMEM((CPW, K), jnp.int32),
        pltpu.VMEM((CPW, K), jnp.int32),
        pltpu.VMEM((CPW, BPW), jnp.float32),
    ],
)(_sc_body)


def _tc_body(m1_ref, m2_ref, p1_ref, p2_ref, out_ref):
    def k_body(kk, acc):
        kbase = kk * TCU
        for u in range(TCU):
            i1 = m1_ref[0, 0, kbase + u]
            i2 = m2_ref[0, 0, kbase + u]
            acc = acc + p1_ref[i1] * p2_ref[i2]
        return acc

    acc = lax.fori_loop(0, K // TCU, k_body,
                        jnp.zeros((8, 128), jnp.float32))
    out_ref[0] = acc


_tc_call = pl.pallas_call(
    _tc_body,
    grid=(CT,),
    in_specs=[
        pl.BlockSpec((1, 1, K), lambda c: (c, 0, 0),
                     memory_space=pltpu.SMEM),
        pl.BlockSpec((1, 1, K), lambda c: (c, 0, 0),
                     memory_space=pltpu.SMEM),
        pl.BlockSpec((C, 8, 128), lambda c: (0, 0, 0)),
        pl.BlockSpec((C, 8, 128), lambda c: (0, 0, 0)),
    ],
    out_specs=pl.BlockSpec((1, 8, 128), lambda c: (c, 0, 0)),
    out_shape=jax.ShapeDtypeStruct((CT, 8, 128), jnp.float32),
    compiler_params=pltpu.CompilerParams(
        dimension_semantics=("arbitrary",),
    ),
)


def kernel(pred1, pred2, mapping1, mapping2):
    m1 = mapping1.astype(jnp.int32)
    m2 = mapping2.astype(jnp.int32)

    p1t = pred1.T.astype(jnp.bfloat16)
    p2t = pred2.T.astype(jnp.bfloat16)
    p1p = (lax.bitcast_convert_type(p1t.reshape(C, B // 2, 2), jnp.int32)
           .reshape(C, NG, WPW).transpose(1, 0, 2))
    p2p = (lax.bitcast_convert_type(p2t.reshape(C, B // 2, 2), jnp.int32)
           .reshape(C, NG, WPW).transpose(1, 0, 2))
    sc_out = _sc_call(p1p, p2p, m1[CT:], m2[CT:])
    sc_fixed = (sc_out.reshape(CS, B // 32, 2, 16)
                .transpose(0, 1, 3, 2)
                .reshape(CS, B))

    p1r = pred1.T.reshape(C, 8, 128)
    p2r = pred2.T.reshape(C, 8, 128)
    tc_out = _tc_call(m1[:CT].reshape(CT, 1, K),
                      m2[:CT].reshape(CT, 1, K),
                      p1r, p2r).reshape(CT, B)

    out_t = jnp.concatenate([tc_out, sc_fixed], axis=0)
    return out_t.T


# speedup vs baseline: 1.2869x; 1.2869x over previous
import functools

import jax
import jax.numpy as jnp
from jax import lax
from jax.experimental import pallas as pl
from jax.experimental.pallas import tpu as pltpu
from jax.experimental.pallas import tpu_sc as plsc

B = 1024
C = 256
K = 256
CT = 96
CS = C - CT
NW = 32
NG = 8
BPW = B // NG
NB = 3
NF = 2
WB = NB * 16
WF = NF * 16
NQ = NW // NG
CPW = CS // NQ
KU = 16
TCU = 8

_HIMASK = -65536


def _widen(word):
    lo = lax.bitcast_convert_type(lax.shift_left(word, 16), jnp.float32)
    hi = lax.bitcast_convert_type(
        lax.bitwise_and(word, jnp.int32(_HIMASK)), jnp.float32)
    return lo, hi


def _sc_body(p1_hbm, p2_hbm, m12_hbm, out_hbm,
             p1_v, p2_v, m12_v, acc_v):
    cid = lax.axis_index("c")
    sid = lax.axis_index("s")
    wid = sid * 2 + cid
    g = lax.rem(wid, NG)
    q = lax.div(wid, NG)
    b0 = g * BPW
    c0 = q * CPW

    pltpu.sync_copy(p1_hbm.at[g], p1_v)
    pltpu.sync_copy(p2_hbm.at[g], p2_v)
    pltpu.sync_copy(m12_hbm.at[pl.ds(c0, CPW), :], m12_v)

    def c_body(ci, carry):
        def k_body(kk, accs):
            accs = list(accs)
            kbase = kk * KU
            iv = m12_v[ci, pl.ds(kbase, KU)]
            for u in range(KU):
                i12 = iv[u]
                i1 = lax.bitwise_and(i12, jnp.int32(0xFFFF))
                i2 = lax.shift_right_logical(i12, 16)
                for h in range(NB):
                    a_lo, a_hi = _widen(p1_v[i1, pl.ds(h * 16, 16)])
                    b_lo, b_hi = _widen(p2_v[i2, pl.ds(h * 16, 16)])
                    accs[2 * h] = accs[2 * h] + a_lo * b_lo
                    accs[2 * h + 1] = accs[2 * h + 1] + a_hi * b_hi
                for j in range(NF):
                    fa = lax.bitcast_convert_type(
                        p1_v[i1, pl.ds(WB + j * 16, 16)], jnp.float32)
                    fb = lax.bitcast_convert_type(
                        p2_v[i2, pl.ds(WB + j * 16, 16)], jnp.float32)
                    accs[2 * NB + j] = accs[2 * NB + j] + fa * fb
            return tuple(accs)

        z = jnp.zeros((16,), jnp.float32)
        accs = lax.fori_loop(0, K // KU, k_body, (z,) * (2 * NB + NF))
        for h in range(NB):
            acc_v[ci, pl.ds(h * 32, 16)] = accs[2 * h]
            acc_v[ci, pl.ds(h * 32 + 16, 16)] = accs[2 * h + 1]
        for j in range(NF):
            acc_v[ci, pl.ds(NB * 32 + j * 16, 16)] = accs[2 * NB + j]
        return carry

    lax.fori_loop(0, CPW, c_body, 0)
    pltpu.sync_copy(acc_v, out_hbm.at[pl.ds(c0, CPW), pl.ds(b0, BPW)])


_sc_call = functools.partial(
    pl.kernel,
    mesh=plsc.VectorSubcoreMesh(core_axis_name="c", subcore_axis_name="s"),
    out_type=jax.ShapeDtypeStruct((CS, B), jnp.float32),
    scratch_types=[
        pltpu.VMEM((C, WB + WF), jnp.int32),
        pltpu.VMEM((C, WB + WF), jnp.int32),
        pltpu.VMEM((CPW, K), jnp.int32),
        pltpu.VMEM((CPW, BPW), jnp.float32),
    ],
)(_sc_body)


def _tc_body(m1_ref, m2_ref, p1_ref, p2_ref, out_ref):
    def k_body(kk, acc):
        kbase = kk * TCU
        for u in range(TCU):
            i1 = m1_ref[0, 0, kbase + u]
            i2 = m2_ref[0, 0, kbase + u]
            acc = acc + p1_ref[i1] * p2_ref[i2]
        return acc

    acc = lax.fori_loop(0, K // TCU, k_body,
                        jnp.zeros((8, 128), jnp.float32))
    out_ref[0] = acc


_tc_call = pl.pallas_call(
    _tc_body,
    grid=(CT,),
    in_specs=[
        pl.BlockSpec((1, 1, K), lambda c: (c, 0, 0),
                     memory_space=pltpu.SMEM),
        pl.BlockSpec((1, 1, K), lambda c: (c, 0, 0),
                     memory_space=pltpu.SMEM),
        pl.BlockSpec((C, 8, 128), lambda c: (0, 0, 0)),
        pl.BlockSpec((C, 8, 128), lambda c: (0, 0, 0)),
    ],
    out_specs=pl.BlockSpec((1, 8, 128), lambda c: (c, 0, 0)),
    out_shape=jax.ShapeDtypeStruct((CT, 8, 128), jnp.float32),
    compiler_params=pltpu.CompilerParams(
        dimension_semantics=("arbitrary",),
    ),
)


def _sc_tables(pt):
    packed = lax.bitcast_convert_type(
        pt.astype(jnp.bfloat16).reshape(C, B // 2, 2), jnp.int32)
    pb = packed.reshape(C, NG, BPW // 2)[:, :, :WB]
    pf = lax.bitcast_convert_type(
        pt.reshape(C, NG, BPW)[:, :, NB * 32:], jnp.int32)
    return jnp.concatenate([pb, pf], axis=2).transpose(1, 0, 2)


def kernel(pred1, pred2, mapping1, mapping2):
    m1 = mapping1.astype(jnp.int32)
    m2 = mapping2.astype(jnp.int32)
    p1t = pred1.T
    p2t = pred2.T

    p1s = _sc_tables(p1t)
    p2s = _sc_tables(p2t)
    m12 = jnp.bitwise_or(m1[CT:], jnp.left_shift(m2[CT:], 16))
    sc_out = _sc_call(p1s, p2s, m12)
    blk = sc_out.reshape(CS, NG, BPW)
    ev = (blk[:, :, :NB * 32].reshape(CS, NG, NB, 2, 16)
          .transpose(0, 1, 2, 4, 3).reshape(CS, NG, NB * 32))
    sc_fixed = jnp.concatenate([ev, blk[:, :, NB * 32:]],
                               axis=2).reshape(CS, B)

    p1r = p1t.reshape(C, 8, 128)
    p2r = p2t.reshape(C, 8, 128)
    tc_out = _tc_call(m1[:CT].reshape(CT, 1, K),
                      m2[:CT].reshape(CT, 1, K),
                      p1r, p2r).reshape(CT, B)

    out_t = jnp.concatenate([tc_out, sc_fixed], axis=0)
    return out_t.T


# speedup vs baseline: 1.2957x; 1.0068x over previous
import functools

import jax
import jax.numpy as jnp
from jax import lax
from jax.experimental import pallas as pl
from jax.experimental.pallas import tpu as pltpu
from jax.experimental.pallas import tpu_sc as plsc

B = 1024
C = 256
K = 256
CT = 96
CS = C - CT
NW = 32
NG = 8
BPW = B // NG
NB = 3
NF = 2
WB = NB * 16
WF = NF * 16
NQ = NW // NG
CPW = CS // NQ
KU = 16
TCU = 8

_HIMASK = -65536


def _widen(word):
    lo = lax.bitcast_convert_type(lax.shift_left(word, 16), jnp.float32)
    hi = lax.bitcast_convert_type(
        lax.bitwise_and(word, jnp.int32(_HIMASK)), jnp.float32)
    return lo, hi


def _sc_body(p1_hbm, p2_hbm, m12_hbm, out_hbm,
             p1_v, p2_v, m12_v, acc_v, dma_sem):
    cid = lax.axis_index("c")
    sid = lax.axis_index("s")
    wid = sid * 2 + cid
    g = lax.rem(wid, NG)
    q = lax.div(wid, NG)
    b0 = g * BPW
    c0 = q * CPW

    cp1 = pltpu.async_copy(p1_hbm.at[g], p1_v, dma_sem)
    cp2 = pltpu.async_copy(p2_hbm.at[g], p2_v, dma_sem)
    cp3 = pltpu.async_copy(m12_hbm.at[pl.ds(c0, CPW), :], m12_v, dma_sem)
    cp1.wait()
    cp2.wait()
    cp3.wait()

    def c_body(ci, carry):
        def k_body(kk, accs):
            accs = list(accs)
            kbase = kk * KU
            iv = m12_v[ci, pl.ds(kbase, KU)]
            for u in range(KU):
                i12 = iv[u]
                i1 = lax.bitwise_and(i12, jnp.int32(0xFFFF))
                i2 = lax.shift_right_logical(i12, 16)
                for h in range(NB):
                    a_lo, a_hi = _widen(p1_v[i1, pl.ds(h * 16, 16)])
                    b_lo, b_hi = _widen(p2_v[i2, pl.ds(h * 16, 16)])
                    accs[2 * h] = accs[2 * h] + a_lo * b_lo
                    accs[2 * h + 1] = accs[2 * h + 1] + a_hi * b_hi
                for j in range(NF):
                    fa = lax.bitcast_convert_type(
                        p1_v[i1, pl.ds(WB + j * 16, 16)], jnp.float32)
                    fb = lax.bitcast_convert_type(
                        p2_v[i2, pl.ds(WB + j * 16, 16)], jnp.float32)
                    accs[2 * NB + j] = accs[2 * NB + j] + fa * fb
            return tuple(accs)

        z = jnp.zeros((16,), jnp.float32)
        accs = lax.fori_loop(0, K // KU, k_body, (z,) * (2 * NB + NF))
        for h in range(NB):
            acc_v[ci, pl.ds(h * 32, 16)] = accs[2 * h]
            acc_v[ci, pl.ds(h * 32 + 16, 16)] = accs[2 * h + 1]
        for j in range(NF):
            acc_v[ci, pl.ds(NB * 32 + j * 16, 16)] = accs[2 * NB + j]
        return carry

    lax.fori_loop(0, CPW, c_body, 0)
    pltpu.sync_copy(acc_v, out_hbm.at[pl.ds(c0, CPW), pl.ds(b0, BPW)])


_sc_call = functools.partial(
    pl.kernel,
    mesh=plsc.VectorSubcoreMesh(core_axis_name="c", subcore_axis_name="s"),
    out_type=jax.ShapeDtypeStruct((CS, B), jnp.float32),
    scratch_types=[
        pltpu.VMEM((C, WB + WF), jnp.int32),
        pltpu.VMEM((C, WB + WF), jnp.int32),
        pltpu.VMEM((CPW, K), jnp.int32),
        pltpu.VMEM((CPW, BPW), jnp.float32),
        pltpu.SemaphoreType.DMA,
    ],
)(_sc_body)


def _tc_body(m1_ref, m2_ref, p1_ref, p2_ref, out_ref):
    def k_body(kk, acc):
        kbase = kk * TCU
        for u in range(TCU):
            i1 = m1_ref[0, 0, kbase + u]
            i2 = m2_ref[0, 0, kbase + u]
            acc = acc + p1_ref[i1] * p2_ref[i2]
        return acc

    acc = lax.fori_loop(0, K // TCU, k_body,
                        jnp.zeros((8, 128), jnp.float32))
    out_ref[0] = acc


_tc_call = pl.pallas_call(
    _tc_body,
    grid=(CT,),
    in_specs=[
        pl.BlockSpec((1, 1, K), lambda c: (c, 0, 0),
                     memory_space=pltpu.SMEM),
        pl.BlockSpec((1, 1, K), lambda c: (c, 0, 0),
                     memory_space=pltpu.SMEM),
        pl.BlockSpec((C, 8, 128), lambda c: (0, 0, 0)),
        pl.BlockSpec((C, 8, 128), lambda c: (0, 0, 0)),
    ],
    out_specs=pl.BlockSpec((1, 8, 128), lambda c: (c, 0, 0)),
    out_shape=jax.ShapeDtypeStruct((CT, 8, 128), jnp.float32),
    compiler_params=pltpu.CompilerParams(
        dimension_semantics=("arbitrary",),
    ),
)


def _sc_tables(pt):
    packed = lax.bitcast_convert_type(
        pt.astype(jnp.bfloat16).reshape(C, B // 2, 2), jnp.int32)
    pb = packed.reshape(C, NG, BPW // 2)[:, :, :WB]
    pf = lax.bitcast_convert_type(
        pt.reshape(C, NG, BPW)[:, :, NB * 32:], jnp.int32)
    return jnp.concatenate([pb, pf], axis=2).transpose(1, 0, 2)


def kernel(pred1, pred2, mapping1, mapping2):
    m1 = mapping1.astype(jnp.int32)
    m2 = mapping2.astype(jnp.int32)
    p1t = pred1.T
    p2t = pred2.T

    p1s = _sc_tables(p1t)
    p2s = _sc_tables(p2t)
    m12 = jnp.bitwise_or(m1[CT:], jnp.left_shift(m2[CT:], 16))
    sc_out = _sc_call(p1s, p2s, m12)
    blk = sc_out.reshape(CS, NG, BPW)
    ev = (blk[:, :, :NB * 32].reshape(CS, NG, NB, 2, 16)
          .transpose(0, 1, 2, 4, 3).reshape(CS, NG, NB * 32))
    sc_fixed = jnp.concatenate([ev, blk[:, :, NB * 32:]],
                               axis=2).reshape(CS, B)

    p1r = p1t.reshape(C, 8, 128)
    p2r = p2t.reshape(C, 8, 128)
    tc_out = _tc_call(m1[:CT].reshape(CT, 1, K),
                      m2[:CT].reshape(CT, 1, K),
                      p1r, p2r).reshape(CT, B)

    out_t = jnp.concatenate([tc_out, sc_fixed], axis=0)
    return out_t.T
